# baseline (device time: 21501 ns/iter reference)
import jax
import jax.numpy as jnp
from jax import lax
from jax.experimental import pallas as pl
from jax.experimental.pallas import tpu as pltpu

N_DEV = 16
NZ = 4
BLK = 64
N = 1024
NS = 3
SW = 256
CCOL = NS * SW
PG = NZ * BLK

_GELU_C = 0.7978845608028654
_MESH = pl.DeviceIdType.MESH


def _gelu(v):
    return 0.5 * v * (1.0 + jnp.tanh(_GELU_C * (v + 0.044715 * v * v * v)))


def kernel(x, w_mat):
    m, k_per = x.shape
    _, n = w_mat.shape

    def body(x_ref, w_ref, out_ref, acc_ref,
             p1s, p1r, p2s, p2r, zfull, zdr,
             czs, czr, cp1s, cp1r, cp2s, cp2r,
             p1_ssem, p1_rsem, p2_ssem, p2_rsem, z_ssem, z_rsem,
             cz_ssem, cz_rsem, cp1_ssem, cp1_rsem, cp2_ssem, cp2_rsem):
        p = lax.axis_index("i")
        z = p // NZ
        c = lax.rem(p, NZ)
        cx = jnp.bitwise_xor(c, 1)
        cy = 3 - c
        cd = 3 - cx
        px = NZ * z + cx
        py = NZ * z + cy

        z_dests = []
        for k in range(NZ - 1):
            zd = k + (k >= z).astype(jnp.int32)
            z_dests.append((zd, NZ * zd + c, jnp.where(z > zd, z - 1, z)))

        xb = x_ref[...].astype(jnp.bfloat16)
        wb = w_ref[...].astype(jnp.bfloat16)
        ccols = slice(CCOL, N)

        acc_ref[:, ccols] = jnp.dot(xb, wb[:, ccols],
                                    preferred_element_type=jnp.float32)
        barrier = pltpu.get_barrier_semaphore()
        for nbr in (px, py) + tuple(d for _, d, _ in z_dests):
            pl.semaphore_signal(barrier, inc=1, device_id=(nbr,),
                                device_id_type=_MESH)
        cz_sends = []
        for k, (zd, pzd, slot) in enumerate(z_dests):
            czs[k] = (acc_ref[pl.ds(zd * PG, PG), ccols]
                      .astype(jnp.bfloat16))
        pl.semaphore_wait(barrier, 5)
        for k, (zd, pzd, slot) in enumerate(z_dests):
            r = pltpu.make_async_remote_copy(
                src_ref=czs.at[k], dst_ref=czr.at[slot],
                send_sem=cz_ssem.at[k], recv_sem=cz_rsem.at[slot],
                device_id=(pzd,), device_id_type=_MESH)
            r.start()
            cz_sends.append(r)

        def strip_cfg(s):
            if s % 2 == 0:
                o1, o2 = jnp.minimum(cx, cd), jnp.maximum(cx, cd)
                return (px, py), (o1, o2), cy
            o1, o2 = jnp.minimum(cy, cd), jnp.maximum(cy, cd)
            return (py, px), (o1, o2), cx

        p1_rdmas, p2_rdmas, z_sends = [], [], []

        for s in range(NS):
            cols = slice(s * SW, (s + 1) * SW)
            (peer1, _), (o1, o2), _ = strip_cfg(s)
            acc_ref[:, cols] = jnp.dot(xb, wb[:, cols],
                                       preferred_element_type=jnp.float32)
            for zi in range(NZ):
                base = zi * NZ * BLK
                p1s[s, pl.ds(2 * zi * BLK, BLK), :] = (
                    acc_ref[pl.ds(base + o1 * BLK, BLK), cols]
                    .astype(jnp.bfloat16))
                p1s[s, pl.ds((2 * zi + 1) * BLK, BLK), :] = (
                    acc_ref[pl.ds(base + o2 * BLK, BLK), cols]
                    .astype(jnp.bfloat16))
            r = pltpu.make_async_remote_copy(
                src_ref=p1s.at[s], dst_ref=p1r.at[s],
                send_sem=p1_ssem.at[s], recv_sem=p1_rsem.at[s],
                device_id=(peer1,), device_id_type=_MESH)
            r.start()
            p1_rdmas.append(r)

        for s in range(NS):
            cols = slice(s * SW, (s + 1) * SW)
            (_, peer2), _, fwd = strip_cfg(s)
            j_c = (c > fwd).astype(jnp.int32)
            j_f = 1 - j_c
            p1_rdmas[s].wait()
            for zi in range(NZ):
                base = zi * NZ * BLK
                rc = pl.ds(base + c * BLK, BLK)
                acc_ref[rc, cols] = (
                    acc_ref[rc, cols]
                    + p1r[s, pl.ds((2 * zi + j_c) * BLK, BLK), :]
                    .astype(jnp.float32))
                p2s[s, pl.ds(zi * BLK, BLK), :] = (
                    acc_ref[pl.ds(base + fwd * BLK, BLK), cols]
                    + p1r[s, pl.ds((2 * zi + j_f) * BLK, BLK), :]
                    .astype(jnp.float32)).astype(jnp.bfloat16)
            r = pltpu.make_async_remote_copy(
                src_ref=p2s.at[s], dst_ref=p2r.at[s],
                send_sem=p2_ssem.at[s], recv_sem=p2_rsem.at[s],
                device_id=(peer2,), device_id_type=_MESH)
            r.start()
            p2_rdmas.append(r)

        for j in range(NZ - 1):
            pltpu.make_async_remote_copy(
                src_ref=czs.at[j], dst_ref=czr.at[j],
                send_sem=cz_ssem.at[j], recv_sem=cz_rsem.at[j],
                device_id=(p,), device_id_type=_MESH).wait_recv()
        mypg = pl.ds(z * PG, PG)
        acc_ref[mypg, ccols] = (
            acc_ref[mypg, ccols]
            + czr[0].astype(jnp.float32)
            + czr[1].astype(jnp.float32)
            + czr[2].astype(jnp.float32))
        co1, co2 = jnp.minimum(cy, cd), jnp.maximum(cy, cd)
        cp1s[pl.ds(0, BLK), :] = (
            acc_ref[pl.ds(z * PG + co1 * BLK, BLK), ccols]
            .astype(jnp.bfloat16))
        cp1s[pl.ds(BLK, BLK), :] = (
            acc_ref[pl.ds(z * PG + co2 * BLK, BLK), ccols]
            .astype(jnp.bfloat16))
        rdma_cp1 = pltpu.make_async_remote_copy(
            src_ref=cp1s, dst_ref=cp1r,
            send_sem=cp1_ssem.at[0], recv_sem=cp1_rsem.at[0],
            device_id=(py,), device_id_type=_MESH)
        rdma_cp1.start()

        for s in range(NS):
            cols = slice(s * SW, (s + 1) * SW)
            p2_rdmas[s].wait()
            for zi in range(NZ):
                rc = pl.ds((zi * NZ + c) * BLK, BLK)
                zfull[zi, :, pl.ds(s * SW, SW)] = (
                    acc_ref[rc, cols]
                    + p2r[s, pl.ds(zi * BLK, BLK), :].astype(jnp.float32)
                ).astype(jnp.bfloat16)
            for k, (zd, pzd, slot) in enumerate(z_dests):
                r = pltpu.make_async_remote_copy(
                    src_ref=zfull.at[zd, :, pl.ds(s * SW, SW)],
                    dst_ref=zdr.at[slot, :, pl.ds(s * SW, SW)],
                    send_sem=z_ssem.at[k * NS + s],
                    recv_sem=z_rsem.at[slot * NS + s],
                    device_id=(pzd,), device_id_type=_MESH)
                r.start()
                z_sends.append(r)

        cj_c = (c > cx).astype(jnp.int32)
        cj_f = 1 - cj_c
        rdma_cp1.wait()
        rcm = pl.ds(z * PG + c * BLK, BLK)
        acc_ref[rcm, ccols] = (
            acc_ref[rcm, ccols]
            + cp1r[pl.ds(cj_c * BLK, BLK), :].astype(jnp.float32))
        cp2s[...] = (
            acc_ref[pl.ds(z * PG + cx * BLK, BLK), ccols]
            + cp1r[pl.ds(cj_f * BLK, BLK), :].astype(jnp.float32)
        ).astype(jnp.bfloat16)
        rdma_cp2 = pltpu.make_async_remote_copy(
            src_ref=cp2s, dst_ref=cp2r,
            send_sem=cp2_ssem.at[0], recv_sem=cp2_rsem.at[0],
            device_id=(px,), device_id_type=_MESH)
        rdma_cp2.start()

        for s in range(NS):
            cols = slice(s * SW, (s + 1) * SW)
            fin = zfull[z, :, cols].astype(jnp.float32)
            for j in range(NZ - 1):
                pltpu.make_async_remote_copy(
                    src_ref=zfull.at[j, :, pl.ds(s * SW, SW)],
                    dst_ref=zdr.at[j, :, pl.ds(s * SW, SW)],
                    send_sem=z_ssem.at[j * NS + s],
                    recv_sem=z_rsem.at[j * NS + s],
                    device_id=(p,), device_id_type=_MESH).wait_recv()
                fin = fin + zdr[j, :, cols].astype(jnp.float32)
            out_ref[:, cols] = _gelu(fin)

        rdma_cp2.wait()
        cfin = acc_ref[rcm, ccols] + cp2r[...].astype(jnp.float32)
        out_ref[:, ccols] = _gelu(cfin)

        for r in cz_sends + z_sends:
            r.wait_send()

    return pl.pallas_call(
        body,
        out_shape=jax.ShapeDtypeStruct((BLK, n), jnp.float32),
        in_specs=[
            pl.BlockSpec(memory_space=pltpu.VMEM),
            pl.BlockSpec(memory_space=pltpu.VMEM),
        ],
        out_specs=pl.BlockSpec(memory_space=pltpu.VMEM),
        scratch_shapes=[
            pltpu.VMEM((m, n), jnp.float32),
            pltpu.VMEM((NS, 8 * BLK, SW), jnp.bfloat16),
            pltpu.VMEM((NS, 8 * BLK, SW), jnp.bfloat16),
            pltpu.VMEM((NS, 4 * BLK, SW), jnp.bfloat16),
            pltpu.VMEM((NS, 4 * BLK, SW), jnp.bfloat16),
            pltpu.VMEM((NZ, BLK, CCOL), jnp.bfloat16),
            pltpu.VMEM((NZ - 1, BLK, CCOL), jnp.bfloat16),
            pltpu.VMEM((NZ - 1, PG, N - CCOL), jnp.bfloat16),
            pltpu.VMEM((NZ - 1, PG, N - CCOL), jnp.bfloat16),
            pltpu.VMEM((2 * BLK, N - CCOL), jnp.bfloat16),
            pltpu.VMEM((2 * BLK, N - CCOL), jnp.bfloat16),
            pltpu.VMEM((BLK, N - CCOL), jnp.bfloat16),
            pltpu.VMEM((BLK, N - CCOL), jnp.bfloat16),
            pltpu.SemaphoreType.DMA((NS,)),
            pltpu.SemaphoreType.DMA((NS,)),
            pltpu.SemaphoreType.DMA((NS,)),
            pltpu.SemaphoreType.DMA((NS,)),
            pltpu.SemaphoreType.DMA(((NZ - 1) * NS,)),
            pltpu.SemaphoreType.DMA(((NZ - 1) * NS,)),
            pltpu.SemaphoreType.DMA((NZ - 1,)),
            pltpu.SemaphoreType.DMA((NZ - 1,)),
            pltpu.SemaphoreType.DMA((1,)),
            pltpu.SemaphoreType.DMA((1,)),
            pltpu.SemaphoreType.DMA((1,)),
            pltpu.SemaphoreType.DMA((1,)),
        ],
        compiler_params=pltpu.CompilerParams(collective_id=0),
    )(x, w_mat)
